# baseline (speedup 1.0000x reference)
import functools

import jax
import jax.numpy as jnp
from jax import lax
from jax.experimental import pallas as pl
from jax.experimental.pallas import tpu as pltpu
from jax.experimental.pallas import tpu_sc as plsc

LANES = 16
SENT = -1


def _k1_body(x_ref, w_ref, asv_ref, adv_ref, h_ref, asrc_ref, adst_ref, amax_ref):
    i = pl.program_id(0)
    h = jnp.dot(x_ref[...], w_ref[...], preferred_element_type=jnp.float32)
    h_ref[...] = h
    a_s = jnp.sum(h * asv_ref[...], axis=1)
    a_d = jnp.sum(h * adv_ref[...], axis=1)
    asrc_ref[0, 0, :] = a_s
    adst_ref[0, 0, :] = a_d
    blk_max = jnp.max(a_s)

    @pl.when(i == 0)
    def _init():
        amax_ref[0, 0] = blk_max

    @pl.when(i > 0)
    def _acc():
        amax_ref[0, 0] = jnp.maximum(amax_ref[0, 0], blk_max)


def _k1(x, W, att_src, att_dst, nblk, rows):
    n, f = x.shape
    return pl.pallas_call(
        _k1_body,
        grid=(nblk,),
        in_specs=[
            pl.BlockSpec((rows, f), lambda i: (i, 0)),
            pl.BlockSpec((f, f), lambda i: (0, 0)),
            pl.BlockSpec((1, f), lambda i: (0, 0)),
            pl.BlockSpec((1, f), lambda i: (0, 0)),
        ],
        out_specs=[
            pl.BlockSpec((rows, f), lambda i: (i, 0)),
            pl.BlockSpec((1, 1, rows), lambda i: (i, 0, 0)),
            pl.BlockSpec((1, 1, rows), lambda i: (i, 0, 0)),
            pl.BlockSpec((1, 1), lambda i: (0, 0), memory_space=pltpu.SMEM),
        ],
        out_shape=[
            jax.ShapeDtypeStruct((n, f), jnp.float32),
            jax.ShapeDtypeStruct((nblk, 1, rows), jnp.float32),
            jax.ShapeDtypeStruct((nblk, 1, rows), jnp.float32),
            jax.ShapeDtypeStruct((1, 1), jnp.float32),
        ],
    )(x, W, att_src.reshape(1, f), att_dst.reshape(1, f))


def _leaky(v):
    return jnp.maximum(v, 0.2 * v)


def _make_k2(n, f, e_tot, ept, blk, nblk):
    mesh = plsc.VectorSubcoreMesh(core_axis_name="c", subcore_axis_name="s")
    nh = n // 2
    cbuf = blk

    @functools.partial(
        pl.kernel,
        mesh=mesh,
        compiler_params=pltpu.CompilerParams(needs_layout_passes=False),
        out_type=[
            jax.ShapeDtypeStruct((2, nh, f), jnp.float32),
            jax.ShapeDtypeStruct((16, 1, n), jnp.float32),
        ],
        scratch_types=[
            pltpu.VMEM((n + LANES,), jnp.float32),
            pltpu.VMEM((n + LANES,), jnp.float32),
            pltpu.VMEM((LANES,), jnp.float32),
            pltpu.VMEM((1, ept), jnp.int32),
            pltpu.VMEM((4, cbuf), jnp.int32),
            pltpu.VMEM((4, cbuf), jnp.int32),
            pltpu.VMEM((4, cbuf), jnp.float32),
            pltpu.VMEM((n + LANES,), jnp.float32),
            pltpu.VMEM((4, cbuf, f), jnp.float32),
            pltpu.SMEM((4,), jnp.int32),
            pltpu.VMEM_SHARED((nh, f), jnp.float32),
            pltpu.SemaphoreType.DMA,
            pltpu.SemaphoreType.DMA,
            pltpu.SemaphoreType.DMA,
            pltpu.SemaphoreType.DMA,
            pltpu.SemaphoreType.DMA,
            pltpu.SemaphoreType.DMA,
            pltpu.SemaphoreType.DMA,
            pltpu.SemaphoreType.DMA,
        ],
    )
    def k2(h_hbm, edge_hbm, a16_hbm, asrc_hbm, adst_hbm,
           numer_hbm, denom_hbm,
           asrc_t, adst_t, a16_t, pk_t, srcb, dstb, exb, den_t, rows_t, cntb,
           acc_sh, g0, g1, g2, g3, s0, s1, s2, s3):
        gsem = (g0, g1, g2, g3)
        ssem = (s0, s1, s2, s3)
        cid = lax.axis_index("c")
        sid = lax.axis_index("s")

        pltpu.sync_copy(edge_hbm.at[sid], pk_t)
        pltpu.sync_copy(asrc_hbm, asrc_t.at[pl.ds(0, n)])
        pltpu.sync_copy(adst_hbm, adst_t.at[pl.ds(0, n)])
        pltpu.sync_copy(a16_hbm, a16_t)

        zeros = jnp.zeros((LANES,), jnp.float32)
        asrc_t[pl.ds(n, LANES)] = zeros
        adst_t[pl.ds(n, LANES)] = zeros

        def _zrow(i, carry):
            for k in range(f // LANES):
                rows_t[0, i, pl.ds(k * LANES, LANES)] = zeros
            return carry

        lax.fori_loop(0, blk, _zrow, 0)

        chunk = (nh // (16 * 8)) * 8
        tail = nh - 16 * chunk
        row0 = sid * chunk
        left = chunk
        off = 0
        while left > 0:
            step = min(left, blk)
            pltpu.sync_copy(rows_t.at[0, pl.ds(0, step)],
                            acc_sh.at[pl.ds(row0 + off, step)])
            left -= step
            off += step
        if tail:
            @pl.when(sid == 0)
            def _ztail():
                pltpu.sync_copy(rows_t.at[0, pl.ds(0, tail)],
                                acc_sh.at[pl.ds(16 * chunk, tail)])

        def _zden(i, carry):
            den_t[pl.ds(i * LANES, LANES)] = zeros
            return carry

        lax.fori_loop(0, (n + LANES) // LANES, _zden, 0)

        plsc.subcore_barrier()

        a16 = a16_t[...]
        lo_row = cid * nh

        sent16 = jnp.full((LANES,), SENT, dtype=jnp.int32)

        def _calc(j, u):
            for k in range(cbuf // LANES):
                sl = pl.ds(k * LANES, LANES)
                srcb[u, sl] = sent16
                dstb[u, sl] = sent16
            cnt = jnp.int32(0)
            for k in range(blk // LANES):
                pk = pk_t[0, pl.ds(j * blk + k * LANES, LANES)]
                src16 = jnp.bitwise_and(pk, 0xFFFF)
                dst16 = jnp.right_shift(pk, 16)
                asv = plsc.load_gather(asrc_t, [src16])
                adv = plsc.load_gather(adst_t, [dst16])
                asd = plsc.load_gather(asrc_t, [dst16])
                e = _leaky(asv + adv)
                hi = _leaky(a16 + adv)
                lo = _leaky(asd + adv)
                shift = 0.5 * (hi + lo)
                ex = jnp.exp(e - shift)
                plsc.addupdate_scatter(den_t, [dst16], ex)
                ldst = dst16 - lo_row
                mine = (ldst >= 0) & (ldst < nh)
                csl = pl.ds(cnt, LANES)
                plsc.store_compressed(srcb.at[u, csl], src16, mask=mine)
                plsc.store_compressed(dstb.at[u, csl], ldst, mask=mine)
                plsc.store_compressed(exb.at[u, csl], ex, mask=mine)
                cnt = cnt + plsc.all_reduce_population_count(mine)[0]
            cntb[u] = cnt

        def _g_desc(u):
            return pltpu.make_async_copy(
                h_hbm.at[plsc.Indices(srcb.at[u], ignored_value=SENT)],
                rows_t.at[u], gsem[u])

        def _s_desc(u):
            return pltpu.make_async_copy(
                rows_t.at[u],
                acc_sh.at[plsc.Indices(dstb.at[u], ignored_value=SENT)],
                ssem[u])

        ones_i = jnp.ones((LANES,), jnp.int32)

        def _scale(u):
            def _body(g, iv):
                for v in range(LANES):
                    i = g * LANES + v
                    s = plsc.load_gather(exb.at[u], [iv])
                    iv = iv + ones_i
                    for k in range(f // LANES):
                        sl = pl.ds(k * LANES, LANES)
                        rows_t[u, i, sl] = rows_t[u, i, sl] * s
                return iv

            ngrp = (cntb[u] + (LANES - 1)) // LANES
            lax.fori_loop(0, ngrp, _body, jnp.zeros((LANES,), jnp.int32))

        _calc(0, 0)
        _g_desc(0).start()
        _calc(1, 1)
        _g_desc(1).start()

        def _step(t, carry):
            for u in range(4):
                j = 4 * t + u
                nx2 = (u + 2) % 4

                @pl.when(j >= 2)
                def _drain():
                    _s_desc(nx2).wait()

                @pl.when(j < nblk - 2)
                def _prefetch():
                    _calc(j + 2, nx2)
                    _g_desc(nx2).start()

                _g_desc(u).wait()
                _scale(u)
                _s_desc(u).start(add=True)
            return carry

        lax.fori_loop(0, nblk // 4, _step, 0)
        _s_desc((nblk - 2) % 4).wait()
        _s_desc((nblk - 1) % 4).wait()

        plsc.subcore_barrier()

        @pl.when(cid == 0)
        def _wden():
            pltpu.sync_copy(den_t.at[pl.ds(0, n)], denom_hbm.at[sid, 0])

        left = chunk
        off = 0
        while left > 0:
            step = min(left, blk)
            pltpu.sync_copy(acc_sh.at[pl.ds(row0 + off, step)],
                            numer_hbm.at[cid, pl.ds(row0 + off, step)])
            left -= step
            off += step
        if tail:
            @pl.when(sid == 0)
            def _wtail():
                pltpu.sync_copy(acc_sh.at[pl.ds(16 * chunk, tail)],
                                numer_hbm.at[cid, pl.ds(16 * chunk, tail)])

    return k2


def _k3_body(num_ref, den_ref, x_ref, bc_ref, w1a_ref, w1b_ref, b1_ref,
             w2_ref, b2_ref, w3_ref, b3_ref, xe_ref, probs_ref):
    i = pl.program_id(0)
    dsum = jnp.sum(den_ref[:, i, :], axis=0) + 1e-16
    xe = num_ref[...] / dsum[:, None] + bc_ref[...]
    xe_ref[...] = xe
    xr = jnp.maximum(xe, 0.0)
    z = jnp.dot(x_ref[...], w1a_ref[...], preferred_element_type=jnp.float32)
    z += jnp.dot(xr, w1b_ref[...], preferred_element_type=jnp.float32)
    z = jnp.maximum(z + b1_ref[...], 0.0)
    z = jnp.dot(z, w2_ref[...], preferred_element_type=jnp.float32)
    z = jnp.maximum(z + b2_ref[...], 0.0)
    z = jnp.dot(z, w3_ref[...], preferred_element_type=jnp.float32)
    z = z + b3_ref[...]
    probs_ref[...] = 1.0 / (1.0 + jnp.exp(-z))


def _k3(num, den, x, bias_conv, W1, b1, W2, b2, W3, b3, nblk, rows):
    n, f = x.shape
    h1 = W1.shape[1]
    h2 = W2.shape[1]
    c = W3.shape[1]
    npart = den.shape[0]
    return pl.pallas_call(
        _k3_body,
        grid=(nblk,),
        in_specs=[
            pl.BlockSpec((rows, f), lambda i: (i, 0)),
            pl.BlockSpec((npart, nblk, rows), lambda i: (0, 0, 0)),
            pl.BlockSpec((rows, f), lambda i: (i, 0)),
            pl.BlockSpec((1, f), lambda i: (0, 0)),
            pl.BlockSpec((f, h1), lambda i: (0, 0)),
            pl.BlockSpec((f, h1), lambda i: (0, 0)),
            pl.BlockSpec((1, h1), lambda i: (0, 0)),
            pl.BlockSpec((h1, h2), lambda i: (0, 0)),
            pl.BlockSpec((1, h2), lambda i: (0, 0)),
            pl.BlockSpec((h2, c), lambda i: (0, 0)),
            pl.BlockSpec((1, c), lambda i: (0, 0)),
        ],
        out_specs=[
            pl.BlockSpec((rows, f), lambda i: (i, 0)),
            pl.BlockSpec((rows, c), lambda i: (i, 0)),
        ],
        out_shape=[
            jax.ShapeDtypeStruct((n, f), jnp.float32),
            jax.ShapeDtypeStruct((n, c), jnp.float32),
        ],
    )(num, den, x, bias_conv.reshape(1, f), W1[:f], W1[f:],
      b1.reshape(1, h1), W2, b2.reshape(1, h2), W3, b3.reshape(1, c))


def kernel(x, edge_index, W, att_src, att_dst, bias_conv, W1, b1, W2, b2, W3, b3):
    n, f = x.shape
    e = edge_index.shape[1]
    e_tot = e + n
    ntile = 16
    blk = 64
    nblk = -(-e_tot // (ntile * blk))
    nblk = 4 * (-(-nblk // 4))
    ept = nblk * blk
    e_pad = ntile * ept

    ei = edge_index.astype(jnp.int32)
    pk_e = (ei[1] << 16) | ei[0]
    pk_loop = jnp.arange(n, dtype=jnp.int32) * 65537
    pk_pad = jnp.full((e_pad - e_tot,), n << 16, dtype=jnp.int32)
    pk3 = jnp.concatenate([pk_e, pk_loop, pk_pad]).reshape(ntile, 1, ept)

    tc_rows = 2000
    tc_nblk = n // tc_rows
    h, asrc_b, adst_b, amax = _k1(x, W, att_src, att_dst, tc_nblk, tc_rows)
    asrc = asrc_b.reshape(n)
    adst = adst_b.reshape(n)
    a16 = jnp.full((LANES,), amax[0, 0], dtype=jnp.float32)

    k2 = _make_k2(n, f, e_tot, ept, blk, nblk)
    numer, denom = k2(h, pk3, a16, asrc, adst)

    return _k3(numer.reshape(n, f), denom.reshape(16, tc_nblk, tc_rows),
               x, bias_conv, W1, b1, W2, b2, W3, b3, tc_nblk, tc_rows)


# speedup vs baseline: 1.0947x; 1.0947x over previous
import functools

import jax
import jax.numpy as jnp
from jax import lax
from jax.experimental import pallas as pl
from jax.experimental.pallas import tpu as pltpu
from jax.experimental.pallas import tpu_sc as plsc

LANES = 16
SENT = -1


def _k1_body(x_ref, w_ref, asv_ref, adv_ref, h_ref, asrc_ref, adst_ref, amax_ref):
    i = pl.program_id(0)
    h = jnp.dot(x_ref[...], w_ref[...], preferred_element_type=jnp.float32)
    h_ref[...] = h
    a_s = jnp.sum(h * asv_ref[...], axis=1)
    a_d = jnp.sum(h * adv_ref[...], axis=1)
    asrc_ref[0, 0, :] = a_s
    adst_ref[0, 0, :] = a_d
    blk_max = jnp.max(a_s)

    @pl.when(i == 0)
    def _init():
        amax_ref[0, 0] = blk_max

    @pl.when(i > 0)
    def _acc():
        amax_ref[0, 0] = jnp.maximum(amax_ref[0, 0], blk_max)


def _k1(x, W, att_src, att_dst, nblk, rows):
    n, f = x.shape
    return pl.pallas_call(
        _k1_body,
        grid=(nblk,),
        in_specs=[
            pl.BlockSpec((rows, f), lambda i: (i, 0)),
            pl.BlockSpec((f, f), lambda i: (0, 0)),
            pl.BlockSpec((1, f), lambda i: (0, 0)),
            pl.BlockSpec((1, f), lambda i: (0, 0)),
        ],
        out_specs=[
            pl.BlockSpec((rows, f), lambda i: (i, 0)),
            pl.BlockSpec((1, 1, rows), lambda i: (i, 0, 0)),
            pl.BlockSpec((1, 1, rows), lambda i: (i, 0, 0)),
            pl.BlockSpec((1, 1), lambda i: (0, 0), memory_space=pltpu.SMEM),
        ],
        out_shape=[
            jax.ShapeDtypeStruct((n, f), jnp.float32),
            jax.ShapeDtypeStruct((nblk, 1, rows), jnp.float32),
            jax.ShapeDtypeStruct((nblk, 1, rows), jnp.float32),
            jax.ShapeDtypeStruct((1, 1), jnp.float32),
        ],
    )(x, W, att_src.reshape(1, f), att_dst.reshape(1, f))


def _leaky(v):
    return jnp.maximum(v, 0.2 * v)


def _make_k2(n, f, e_tot, ept, blk, nblk):
    mesh = plsc.VectorSubcoreMesh(core_axis_name="c", subcore_axis_name="s")
    nh = n // 2
    cbuf = blk

    @functools.partial(
        pl.kernel,
        mesh=mesh,
        compiler_params=pltpu.CompilerParams(needs_layout_passes=False),
        out_type=[
            jax.ShapeDtypeStruct((2, nh, f), jnp.float32),
            jax.ShapeDtypeStruct((16, 1, n), jnp.float32),
        ],
        scratch_types=[
            pltpu.VMEM((n + LANES,), jnp.float32),
            pltpu.VMEM((n + LANES,), jnp.float32),
            pltpu.VMEM((LANES,), jnp.float32),
            pltpu.VMEM((1, ept), jnp.int32),
            pltpu.VMEM((4, cbuf), jnp.int32),
            pltpu.VMEM((4, cbuf), jnp.int32),
            pltpu.VMEM((4, cbuf), jnp.float32),
            pltpu.VMEM((n + LANES,), jnp.float32),
            pltpu.VMEM((4, cbuf, f), jnp.float32),
            pltpu.SMEM((4,), jnp.int32),
            pltpu.VMEM_SHARED((nh, f), jnp.float32),
            pltpu.SemaphoreType.DMA,
            pltpu.SemaphoreType.DMA,
            pltpu.SemaphoreType.DMA,
            pltpu.SemaphoreType.DMA,
            pltpu.SemaphoreType.DMA,
            pltpu.SemaphoreType.DMA,
            pltpu.SemaphoreType.DMA,
            pltpu.SemaphoreType.DMA,
        ],
    )
    def k2(h_hbm, edge_hbm, a16_hbm, asrc_hbm, adst_hbm,
           numer_hbm, denom_hbm,
           asrc_t, adst_t, a16_t, pk_t, srcb, dstb, exb, den_t, rows_t, cntb,
           acc_sh, g0, g1, g2, g3, s0, s1, s2, s3):
        gsem = (g0, g1, g2, g3)
        ssem = (s0, s1, s2, s3)
        cid = lax.axis_index("c")
        sid = lax.axis_index("s")

        pltpu.sync_copy(edge_hbm.at[sid], pk_t)
        pltpu.sync_copy(asrc_hbm, asrc_t.at[pl.ds(0, n)])
        pltpu.sync_copy(adst_hbm, adst_t.at[pl.ds(0, n)])
        pltpu.sync_copy(a16_hbm, a16_t)

        zeros = jnp.zeros((LANES,), jnp.float32)
        asrc_t[pl.ds(n, LANES)] = zeros
        adst_t[pl.ds(n, LANES)] = zeros

        def _zrow(i, carry):
            for k in range(f // LANES):
                rows_t[0, i, pl.ds(k * LANES, LANES)] = zeros
            return carry

        lax.fori_loop(0, blk, _zrow, 0)

        chunk = (nh // (16 * 8)) * 8
        tail = nh - 16 * chunk
        row0 = sid * chunk
        left = chunk
        off = 0
        while left > 0:
            step = min(left, blk)
            pltpu.sync_copy(rows_t.at[0, pl.ds(0, step)],
                            acc_sh.at[pl.ds(row0 + off, step)])
            left -= step
            off += step
        if tail:
            @pl.when(sid == 0)
            def _ztail():
                pltpu.sync_copy(rows_t.at[0, pl.ds(0, tail)],
                                acc_sh.at[pl.ds(16 * chunk, tail)])

        def _zden(i, carry):
            den_t[pl.ds(i * LANES, LANES)] = zeros
            return carry

        lax.fori_loop(0, (n + LANES) // LANES, _zden, 0)

        plsc.subcore_barrier()

        a16 = a16_t[...]
        lo_row = cid * nh

        sent16 = jnp.full((LANES,), SENT, dtype=jnp.int32)

        def _calc(j, u):
            for k in range(cbuf // LANES):
                sl = pl.ds(k * LANES, LANES)
                srcb[u, sl] = sent16
                dstb[u, sl] = sent16
            cnt = jnp.int32(0)
            for k in range(blk // LANES):
                pk = pk_t[0, pl.ds(j * blk + k * LANES, LANES)]
                src16 = jnp.bitwise_and(pk, 0xFFFF)
                dst16 = jnp.right_shift(pk, 16)
                asv = plsc.load_gather(asrc_t, [src16])
                adv = plsc.load_gather(adst_t, [dst16])
                asd = plsc.load_gather(asrc_t, [dst16])
                e = _leaky(asv + adv)
                hi = _leaky(a16 + adv)
                lo = _leaky(asd + adv)
                shift = 0.5 * (hi + lo)
                ex = jnp.exp(e - shift)
                plsc.addupdate_scatter(den_t, [dst16], ex)
                ldst = dst16 - lo_row
                mine = (ldst >= 0) & (ldst < nh)
                csl = pl.ds(cnt, LANES)
                plsc.store_compressed(srcb.at[u, csl], src16, mask=mine)
                plsc.store_compressed(dstb.at[u, csl], ldst, mask=mine)
                plsc.store_compressed(exb.at[u, csl], ex, mask=mine)
                cnt = cnt + plsc.all_reduce_population_count(mine)[0]
            cntb[u] = cnt

        def _g_desc(u):
            return pltpu.make_async_copy(
                h_hbm.at[plsc.Indices(srcb.at[u], ignored_value=SENT)],
                rows_t.at[u], gsem[u])

        def _s_desc(u):
            return pltpu.make_async_copy(
                rows_t.at[u],
                acc_sh.at[plsc.Indices(dstb.at[u], ignored_value=SENT)],
                ssem[u])

        def _scale(u):
            def _body(g, c2):
                exv = exb[u, pl.ds(g * LANES, LANES)]
                for v in range(LANES):
                    i = g * LANES + v
                    s = jnp.full((LANES,), exv[v])
                    for k in range(f // LANES):
                        sl = pl.ds(k * LANES, LANES)
                        rows_t[u, i, sl] = rows_t[u, i, sl] * s
                return c2

            ngrp = (cntb[u] + (LANES - 1)) // LANES
            lax.fori_loop(0, ngrp, _body, 0)

        _calc(0, 0)
        _g_desc(0).start()
        _calc(1, 1)
        _g_desc(1).start()

        def _step(t, carry):
            for u in range(4):
                j = 4 * t + u
                nx2 = (u + 2) % 4

                @pl.when(j >= 2)
                def _drain():
                    _s_desc(nx2).wait()

                @pl.when(j < nblk - 2)
                def _prefetch():
                    _calc(j + 2, nx2)
                    _g_desc(nx2).start()

                _g_desc(u).wait()
                _scale(u)
                _s_desc(u).start(add=True)
            return carry

        lax.fori_loop(0, nblk // 4, _step, 0)
        _s_desc((nblk - 2) % 4).wait()
        _s_desc((nblk - 1) % 4).wait()

        plsc.subcore_barrier()

        @pl.when(cid == 0)
        def _wden():
            pltpu.sync_copy(den_t.at[pl.ds(0, n)], denom_hbm.at[sid, 0])

        left = chunk
        off = 0
        while left > 0:
            step = min(left, blk)
            pltpu.sync_copy(acc_sh.at[pl.ds(row0 + off, step)],
                            numer_hbm.at[cid, pl.ds(row0 + off, step)])
            left -= step
            off += step
        if tail:
            @pl.when(sid == 0)
            def _wtail():
                pltpu.sync_copy(acc_sh.at[pl.ds(16 * chunk, tail)],
                                numer_hbm.at[cid, pl.ds(16 * chunk, tail)])

    return k2


def _k3_body(num_ref, den_ref, x_ref, bc_ref, w1a_ref, w1b_ref, b1_ref,
             w2_ref, b2_ref, w3_ref, b3_ref, xe_ref, probs_ref):
    i = pl.program_id(0)
    dsum = jnp.sum(den_ref[:, i, :], axis=0) + 1e-16
    xe = num_ref[...] / dsum[:, None] + bc_ref[...]
    xe_ref[...] = xe
    xr = jnp.maximum(xe, 0.0)
    z = jnp.dot(x_ref[...], w1a_ref[...], preferred_element_type=jnp.float32)
    z += jnp.dot(xr, w1b_ref[...], preferred_element_type=jnp.float32)
    z = jnp.maximum(z + b1_ref[...], 0.0)
    z = jnp.dot(z, w2_ref[...], preferred_element_type=jnp.float32)
    z = jnp.maximum(z + b2_ref[...], 0.0)
    z = jnp.dot(z, w3_ref[...], preferred_element_type=jnp.float32)
    z = z + b3_ref[...]
    probs_ref[...] = 1.0 / (1.0 + jnp.exp(-z))


def _k3(num, den, x, bias_conv, W1, b1, W2, b2, W3, b3, nblk, rows):
    n, f = x.shape
    h1 = W1.shape[1]
    h2 = W2.shape[1]
    c = W3.shape[1]
    npart = den.shape[0]
    return pl.pallas_call(
        _k3_body,
        grid=(nblk,),
        in_specs=[
            pl.BlockSpec((rows, f), lambda i: (i, 0)),
            pl.BlockSpec((npart, nblk, rows), lambda i: (0, 0, 0)),
            pl.BlockSpec((rows, f), lambda i: (i, 0)),
            pl.BlockSpec((1, f), lambda i: (0, 0)),
            pl.BlockSpec((f, h1), lambda i: (0, 0)),
            pl.BlockSpec((f, h1), lambda i: (0, 0)),
            pl.BlockSpec((1, h1), lambda i: (0, 0)),
            pl.BlockSpec((h1, h2), lambda i: (0, 0)),
            pl.BlockSpec((1, h2), lambda i: (0, 0)),
            pl.BlockSpec((h2, c), lambda i: (0, 0)),
            pl.BlockSpec((1, c), lambda i: (0, 0)),
        ],
        out_specs=[
            pl.BlockSpec((rows, f), lambda i: (i, 0)),
            pl.BlockSpec((rows, c), lambda i: (i, 0)),
        ],
        out_shape=[
            jax.ShapeDtypeStruct((n, f), jnp.float32),
            jax.ShapeDtypeStruct((n, c), jnp.float32),
        ],
    )(num, den, x, bias_conv.reshape(1, f), W1[:f], W1[f:],
      b1.reshape(1, h1), W2, b2.reshape(1, h2), W3, b3.reshape(1, c))


def kernel(x, edge_index, W, att_src, att_dst, bias_conv, W1, b1, W2, b2, W3, b3):
    n, f = x.shape
    e = edge_index.shape[1]
    e_tot = e + n
    ntile = 16
    blk = 64
    nblk = -(-e_tot // (ntile * blk))
    nblk = 4 * (-(-nblk // 4))
    ept = nblk * blk
    e_pad = ntile * ept

    ei = edge_index.astype(jnp.int32)
    pk_e = (ei[1] << 16) | ei[0]
    pk_loop = jnp.arange(n, dtype=jnp.int32) * 65537
    pk_pad = jnp.full((e_pad - e_tot,), n << 16, dtype=jnp.int32)
    pk3 = jnp.concatenate([pk_e, pk_loop, pk_pad]).reshape(ntile, 1, ept)

    tc_rows = 2000
    tc_nblk = n // tc_rows
    h, asrc_b, adst_b, amax = _k1(x, W, att_src, att_dst, tc_nblk, tc_rows)
    asrc = asrc_b.reshape(n)
    adst = adst_b.reshape(n)
    a16 = jnp.full((LANES,), amax[0, 0], dtype=jnp.float32)

    k2 = _make_k2(n, f, e_tot, ept, blk, nblk)
    numer, denom = k2(h, pk3, a16, asrc, adst)

    return _k3(numer.reshape(n, f), denom.reshape(16, tc_nblk, tc_rows),
               x, bias_conv, W1, b1, W2, b2, W3, b3, tc_nblk, tc_rows)


# speedup vs baseline: 1.1208x; 1.0239x over previous
import functools

import jax
import jax.numpy as jnp
from jax import lax
from jax.experimental import pallas as pl
from jax.experimental.pallas import tpu as pltpu
from jax.experimental.pallas import tpu_sc as plsc

LANES = 16
SENT = -1


def _k1_body(x_ref, w_ref, asv_ref, adv_ref, h_ref, asrc_ref, adst_ref, amax_ref):
    i = pl.program_id(0)
    h = jnp.dot(x_ref[...], w_ref[...], preferred_element_type=jnp.float32)
    h_ref[...] = h
    a_s = jnp.sum(h * asv_ref[...], axis=1)
    a_d = jnp.sum(h * adv_ref[...], axis=1)
    asrc_ref[0, 0, :] = a_s
    adst_ref[0, 0, :] = a_d
    blk_max = jnp.max(a_s)

    @pl.when(i == 0)
    def _init():
        amax_ref[0, 0] = blk_max

    @pl.when(i > 0)
    def _acc():
        amax_ref[0, 0] = jnp.maximum(amax_ref[0, 0], blk_max)


def _k1(x, W, att_src, att_dst, nblk, rows):
    n, f = x.shape
    return pl.pallas_call(
        _k1_body,
        grid=(nblk,),
        in_specs=[
            pl.BlockSpec((rows, f), lambda i: (i, 0)),
            pl.BlockSpec((f, f), lambda i: (0, 0)),
            pl.BlockSpec((1, f), lambda i: (0, 0)),
            pl.BlockSpec((1, f), lambda i: (0, 0)),
        ],
        out_specs=[
            pl.BlockSpec((rows, f), lambda i: (i, 0)),
            pl.BlockSpec((1, 1, rows), lambda i: (i, 0, 0)),
            pl.BlockSpec((1, 1, rows), lambda i: (i, 0, 0)),
            pl.BlockSpec((1, 1), lambda i: (0, 0), memory_space=pltpu.SMEM),
        ],
        out_shape=[
            jax.ShapeDtypeStruct((n, f), jnp.float32),
            jax.ShapeDtypeStruct((nblk, 1, rows), jnp.float32),
            jax.ShapeDtypeStruct((nblk, 1, rows), jnp.float32),
            jax.ShapeDtypeStruct((1, 1), jnp.float32),
        ],
    )(x, W, att_src.reshape(1, f), att_dst.reshape(1, f))


def _leaky(v):
    return jnp.maximum(v, 0.2 * v)


def _make_k2(n, f, e_tot, ept, blk, nblk):
    mesh = plsc.VectorSubcoreMesh(core_axis_name="c", subcore_axis_name="s")
    nh = n // 2
    cbuf = blk

    @functools.partial(
        pl.kernel,
        mesh=mesh,
        compiler_params=pltpu.CompilerParams(needs_layout_passes=False),
        out_type=[
            jax.ShapeDtypeStruct((2, nh, f), jnp.float32),
            jax.ShapeDtypeStruct((16, 1, n), jnp.float32),
        ],
        scratch_types=[
            pltpu.VMEM((n + LANES,), jnp.float32),
            pltpu.VMEM((n + LANES,), jnp.float32),
            pltpu.VMEM((LANES,), jnp.float32),
            pltpu.VMEM((1, ept), jnp.int32),
            pltpu.VMEM((4, cbuf), jnp.int32),
            pltpu.VMEM((4, cbuf), jnp.int32),
            pltpu.VMEM((4, cbuf), jnp.float32),
            pltpu.VMEM((n + LANES,), jnp.float32),
            pltpu.VMEM((4, cbuf, f), jnp.float32),
            pltpu.SMEM((4,), jnp.int32),
            pltpu.VMEM_SHARED((nh, f), jnp.float32),
            pltpu.SemaphoreType.DMA,
            pltpu.SemaphoreType.DMA,
            pltpu.SemaphoreType.DMA,
            pltpu.SemaphoreType.DMA,
            pltpu.SemaphoreType.DMA,
            pltpu.SemaphoreType.DMA,
            pltpu.SemaphoreType.DMA,
            pltpu.SemaphoreType.DMA,
        ],
    )
    def k2(h_hbm, edge_hbm, a16_hbm, asrc_hbm, adst_hbm,
           numer_hbm, denom_hbm,
           asrc_t, adst_t, a16_t, pk_t, srcb, dstb, exb, den_t, rows_t, cntb,
           acc_sh, g0, g1, g2, g3, s0, s1, s2, s3):
        gsem = (g0, g1, g2, g3)
        ssem = (s0, s1, s2, s3)
        cid = lax.axis_index("c")
        sid = lax.axis_index("s")

        stage = [
            pltpu.async_copy(edge_hbm.at[sid], pk_t, g0),
            pltpu.async_copy(asrc_hbm, asrc_t.at[pl.ds(0, n)], g1),
            pltpu.async_copy(adst_hbm, adst_t.at[pl.ds(0, n)], g2),
            pltpu.async_copy(a16_hbm, a16_t, g3),
        ]

        zeros = jnp.zeros((LANES,), jnp.float32)

        def _zrow(i, carry):
            for k in range(f // LANES):
                rows_t[0, i, pl.ds(k * LANES, LANES)] = zeros
            return carry

        lax.fori_loop(0, blk, _zrow, 0)

        chunk = (nh // (16 * 8)) * 8
        tail = nh - 16 * chunk
        row0 = sid * chunk
        left = chunk
        off = 0
        while left > 0:
            step = min(left, blk)
            pltpu.sync_copy(rows_t.at[0, pl.ds(0, step)],
                            acc_sh.at[pl.ds(row0 + off, step)])
            left -= step
            off += step
        if tail:
            @pl.when(sid == 0)
            def _ztail():
                pltpu.sync_copy(rows_t.at[0, pl.ds(0, tail)],
                                acc_sh.at[pl.ds(16 * chunk, tail)])

        def _zden(i, carry):
            den_t[pl.ds(i * LANES, LANES)] = zeros
            return carry

        lax.fori_loop(0, (n + LANES) // LANES, _zden, 0)

        for cp in stage:
            cp.wait()
        asrc_t[pl.ds(n, LANES)] = zeros
        adst_t[pl.ds(n, LANES)] = zeros

        plsc.subcore_barrier()

        a16 = a16_t[...]
        lo_row = cid * nh

        sent16 = jnp.full((LANES,), SENT, dtype=jnp.int32)

        def _calc(j, u):
            for k in range(cbuf // LANES):
                sl = pl.ds(k * LANES, LANES)
                srcb[u, sl] = sent16
                dstb[u, sl] = sent16
            cnt = jnp.int32(0)
            for k in range(blk // LANES):
                pk = pk_t[0, pl.ds(j * blk + k * LANES, LANES)]
                src16 = jnp.bitwise_and(pk, 0xFFFF)
                dst16 = jnp.right_shift(pk, 16)
                asv = plsc.load_gather(asrc_t, [src16])
                adv = plsc.load_gather(adst_t, [dst16])
                asd = plsc.load_gather(asrc_t, [dst16])
                e = _leaky(asv + adv)
                hi = _leaky(a16 + adv)
                lo = _leaky(asd + adv)
                shift = 0.5 * (hi + lo)
                ex = jnp.exp(e - shift)
                plsc.addupdate_scatter(den_t, [dst16], ex)
                ldst = dst16 - lo_row
                mine = (ldst >= 0) & (ldst < nh)
                csl = pl.ds(cnt, LANES)
                plsc.store_compressed(srcb.at[u, csl], src16, mask=mine)
                plsc.store_compressed(dstb.at[u, csl], ldst, mask=mine)
                plsc.store_compressed(exb.at[u, csl], ex, mask=mine)
                cnt = cnt + plsc.all_reduce_population_count(mine)[0]
            cntb[u] = cnt

        def _g_desc(u):
            return pltpu.make_async_copy(
                h_hbm.at[plsc.Indices(srcb.at[u], ignored_value=SENT)],
                rows_t.at[u], gsem[u])

        def _s_desc(u):
            return pltpu.make_async_copy(
                rows_t.at[u],
                acc_sh.at[plsc.Indices(dstb.at[u], ignored_value=SENT)],
                ssem[u])

        def _scale(u):
            def _body(g, c2):
                exv = exb[u, pl.ds(g * LANES, LANES)]
                for v in range(LANES):
                    i = g * LANES + v
                    s = jnp.full((LANES,), exv[v])
                    for k in range(f // LANES):
                        sl = pl.ds(k * LANES, LANES)
                        rows_t[u, i, sl] = rows_t[u, i, sl] * s
                return c2

            ngrp = (cntb[u] + (LANES - 1)) // LANES
            lax.fori_loop(0, ngrp, _body, 0)

        _calc(0, 0)
        _g_desc(0).start()
        _calc(1, 1)
        _g_desc(1).start()

        def _step(t, carry):
            for u in range(4):
                j = 4 * t + u
                nx2 = (u + 2) % 4

                @pl.when(j >= 2)
                def _drain():
                    _s_desc(nx2).wait()

                @pl.when(j < nblk - 2)
                def _prefetch():
                    _calc(j + 2, nx2)
                    _g_desc(nx2).start()

                _g_desc(u).wait()
                _scale(u)
                _s_desc(u).start(add=True)
            return carry

        lax.fori_loop(0, nblk // 4, _step, 0)
        _s_desc((nblk - 2) % 4).wait()
        _s_desc((nblk - 1) % 4).wait()

        plsc.subcore_barrier()

        @pl.when(cid == 0)
        def _wden():
            pltpu.sync_copy(den_t.at[pl.ds(0, n)], denom_hbm.at[sid, 0])

        left = chunk
        off = 0
        while left > 0:
            step = min(left, blk)
            pltpu.sync_copy(acc_sh.at[pl.ds(row0 + off, step)],
                            numer_hbm.at[cid, pl.ds(row0 + off, step)])
            left -= step
            off += step
        if tail:
            @pl.when(sid == 0)
            def _wtail():
                pltpu.sync_copy(acc_sh.at[pl.ds(16 * chunk, tail)],
                                numer_hbm.at[cid, pl.ds(16 * chunk, tail)])

    return k2


def _k3_body(num_ref, den_ref, x_ref, bc_ref, w1a_ref, w1b_ref, b1_ref,
             w2_ref, b2_ref, w3_ref, b3_ref, xe_ref, probs_ref):
    i = pl.program_id(0)
    dsum = jnp.sum(den_ref[:, i, :], axis=0) + 1e-16
    xe = num_ref[...] / dsum[:, None] + bc_ref[...]
    xe_ref[...] = xe
    xr = jnp.maximum(xe, 0.0)
    z = jnp.dot(x_ref[...], w1a_ref[...], preferred_element_type=jnp.float32)
    z += jnp.dot(xr, w1b_ref[...], preferred_element_type=jnp.float32)
    z = jnp.maximum(z + b1_ref[...], 0.0)
    z = jnp.dot(z, w2_ref[...], preferred_element_type=jnp.float32)
    z = jnp.maximum(z + b2_ref[...], 0.0)
    z = jnp.dot(z, w3_ref[...], preferred_element_type=jnp.float32)
    z = z + b3_ref[...]
    probs_ref[...] = 1.0 / (1.0 + jnp.exp(-z))


def _k3(num, den, x, bias_conv, W1, b1, W2, b2, W3, b3, nblk, rows):
    n, f = x.shape
    h1 = W1.shape[1]
    h2 = W2.shape[1]
    c = W3.shape[1]
    npart = den.shape[0]
    return pl.pallas_call(
        _k3_body,
        grid=(nblk,),
        in_specs=[
            pl.BlockSpec((rows, f), lambda i: (i, 0)),
            pl.BlockSpec((npart, nblk, rows), lambda i: (0, 0, 0)),
            pl.BlockSpec((rows, f), lambda i: (i, 0)),
            pl.BlockSpec((1, f), lambda i: (0, 0)),
            pl.BlockSpec((f, h1), lambda i: (0, 0)),
            pl.BlockSpec((f, h1), lambda i: (0, 0)),
            pl.BlockSpec((1, h1), lambda i: (0, 0)),
            pl.BlockSpec((h1, h2), lambda i: (0, 0)),
            pl.BlockSpec((1, h2), lambda i: (0, 0)),
            pl.BlockSpec((h2, c), lambda i: (0, 0)),
            pl.BlockSpec((1, c), lambda i: (0, 0)),
        ],
        out_specs=[
            pl.BlockSpec((rows, f), lambda i: (i, 0)),
            pl.BlockSpec((rows, c), lambda i: (i, 0)),
        ],
        out_shape=[
            jax.ShapeDtypeStruct((n, f), jnp.float32),
            jax.ShapeDtypeStruct((n, c), jnp.float32),
        ],
    )(num, den, x, bias_conv.reshape(1, f), W1[:f], W1[f:],
      b1.reshape(1, h1), W2, b2.reshape(1, h2), W3, b3.reshape(1, c))


def kernel(x, edge_index, W, att_src, att_dst, bias_conv, W1, b1, W2, b2, W3, b3):
    n, f = x.shape
    e = edge_index.shape[1]
    e_tot = e + n
    ntile = 16
    blk = 64
    nblk = -(-e_tot // (ntile * blk))
    nblk = 4 * (-(-nblk // 4))
    ept = nblk * blk
    e_pad = ntile * ept

    ei = edge_index.astype(jnp.int32)
    pk_e = (ei[1] << 16) | ei[0]
    pk_loop = jnp.arange(n, dtype=jnp.int32) * 65537
    pk_pad = jnp.full((e_pad - e_tot,), n << 16, dtype=jnp.int32)
    pk3 = jnp.concatenate([pk_e, pk_loop, pk_pad]).reshape(ntile, 1, ept)

    tc_rows = 2000
    tc_nblk = n // tc_rows
    h, asrc_b, adst_b, amax = _k1(x, W, att_src, att_dst, tc_nblk, tc_rows)
    asrc = asrc_b.reshape(n)
    adst = adst_b.reshape(n)
    a16 = jnp.full((LANES,), amax[0, 0], dtype=jnp.float32)

    k2 = _make_k2(n, f, e_tot, ept, blk, nblk)
    numer, denom = k2(h, pk3, a16, asrc, adst)

    return _k3(numer.reshape(n, f), denom.reshape(16, tc_nblk, tc_rows),
               x, bias_conv, W1, b1, W2, b2, W3, b3, tc_nblk, tc_rows)
